# Initial kernel scaffold; baseline (speedup 1.0000x reference)
import functools

import jax
import jax.numpy as jnp
from jax import lax
from jax.experimental import pallas as pl
from jax.experimental.pallas import tpu as pltpu
from jax.experimental.pallas import tpu_sc as plsc

N = 10000
E = 320000
D = 128
HALF = 64
NSUB = 16
CHUNK = 80
ROWS_PER_SUB = E // NSUB // CHUNK
EDGE_ROWS = E // CHUNK
NODE_PER_SUB = N // NSUB
WB = 125

_mesh = plsc.VectorSubcoreMesh(core_axis_name="c", subcore_axis_name="s")


@functools.partial(
    pl.kernel,
    mesh=_mesh,
    out_type=jax.ShapeDtypeStruct((2, N, HALF), jnp.float32),
    scratch_types=[
        pltpu.VMEM((ROWS_PER_SUB, CHUNK), jnp.int32),
        pltpu.VMEM((ROWS_PER_SUB, CHUNK), jnp.int32),
        pltpu.VMEM((CHUNK, HALF), jnp.float32),
        pltpu.VMEM((CHUNK, HALF), jnp.float32),
        pltpu.VMEM((WB, HALF), jnp.float32),
        pltpu.VMEM_SHARED((N, HALF), jnp.float32),
        pltpu.SemaphoreType.DMA,
        pltpu.SemaphoreType.DMA,
    ],
)
def _sc_agg(xflat, gsrc, dstr, out, gidx, didx, buf0, buf1, bounce, acc, sem0, sem1):
    c = lax.axis_index("c")
    s = lax.axis_index("s")

    pltpu.sync_copy(gsrc.at[c, pl.ds(s * ROWS_PER_SUB, ROWS_PER_SUB)], gidx)
    pltpu.sync_copy(dstr.at[pl.ds(s * ROWS_PER_SUB, ROWS_PER_SUB)], didx)

    for k in range(NODE_PER_SUB // WB):
        r0 = s * NODE_PER_SUB + k * WB
        pltpu.sync_copy(xflat.at[pl.ds(c * N + r0, WB)], bounce)
        pltpu.sync_copy(bounce, acc.at[pl.ds(r0, WB)])
    plsc.subcore_barrier()

    bufs = (buf0, buf1)
    sems = (sem0, sem1)

    def start(j, b):
        pltpu.make_async_copy(xflat.at[gidx.at[j]], bufs[b], sems[b]).start()

    def wait(j, b):
        pltpu.make_async_copy(xflat.at[gidx.at[j]], bufs[b], sems[b]).wait()

    start(0, 0)
    start(1, 1)

    @pl.loop(0, ROWS_PER_SUB, step=2)
    def _edge_loop(j):
        for b in range(2):
            jj = j + b
            wait(jj, b)
            pltpu.sync_copy(bufs[b], acc.at[didx.at[jj]], add=True)

            @pl.when(jj + 2 < ROWS_PER_SUB)
            def _():
                start(jj + 2, b)

    plsc.subcore_barrier()

    for k in range(NODE_PER_SUB // WB):
        r0 = s * NODE_PER_SUB + k * WB
        pltpu.sync_copy(acc.at[pl.ds(r0, WB)], bounce)
        pltpu.sync_copy(bounce, out.at[c, pl.ds(r0, WB)])


BLK = 2000


def _mlp_body(eps_ref, x_ref, lo_ref, hi_ref, w1_ref, b1_ref, w2_ref, b2_ref,
              w3_ref, b3_ref, w4_ref, b4_ref, o_ref):
    e = eps_ref[0]
    hp = jnp.concatenate([lo_ref[...], hi_ref[...]], axis=1)
    h = hp + e * x_ref[...]
    h1 = jnp.maximum(
        jnp.dot(h, w1_ref[...], preferred_element_type=jnp.float32) + b1_ref[...], 0.0)
    h2 = jnp.dot(h1, w2_ref[...], preferred_element_type=jnp.float32) + b2_ref[...]
    h3 = jnp.maximum(
        jnp.dot(h2, w3_ref[...], preferred_element_type=jnp.float32) + b3_ref[...], 0.0)
    h4 = jnp.dot(h3, w4_ref[...], preferred_element_type=jnp.float32) + b4_ref[...]
    o_ref[...] = 1.0 / (1.0 + jnp.exp(-h4))


def _mlp(eps1, x, lo, hi, W1, b1, W2, b2, W3, b3, W4, b4):
    row_spec = pl.BlockSpec((BLK, D), lambda i: (i, 0))
    half_spec = pl.BlockSpec((BLK, HALF), lambda i: (i, 0))
    w_spec = pl.BlockSpec((D, D), lambda i: (0, 0))
    b_spec = pl.BlockSpec((1, D), lambda i: (0, 0))
    return pl.pallas_call(
        _mlp_body,
        grid=(N // BLK,),
        in_specs=[
            pl.BlockSpec(memory_space=pltpu.SMEM),
            row_spec, half_spec, half_spec,
            w_spec, b_spec, w_spec, b_spec,
            w_spec, b_spec, w_spec, b_spec,
        ],
        out_specs=row_spec,
        out_shape=jax.ShapeDtypeStruct((N, D), jnp.float32),
    )(eps1, x, lo, hi, W1, b1, W2, b2, W3, b3, W4, b4)


def kernel(x, edge_index, eps, W1, b1, W2, b2, W3, b3, W4, b4):
    ei = edge_index.astype(jnp.int32)
    src = ei[0]
    dst = ei[1]
    gsrc = jnp.stack([src, src + N]).reshape(2, EDGE_ROWS, CHUNK)
    dstr = dst.reshape(EDGE_ROWS, CHUNK)
    xflat = x.reshape(N, 2, HALF).transpose(1, 0, 2).reshape(2 * N, HALF)
    hpart = _sc_agg(xflat, gsrc, dstr)
    eps1 = jnp.reshape(eps, (1,)).astype(jnp.float32)
    return _mlp(eps1, x, hpart[0], hpart[1],
                W1, b1.reshape(1, D), W2, b2.reshape(1, D),
                W3, b3.reshape(1, D), W4, b4.reshape(1, D))


# speedup vs baseline: 8.0542x; 8.0542x over previous
import functools

import jax
import jax.numpy as jnp
from jax import lax
from jax.experimental import pallas as pl
from jax.experimental.pallas import tpu as pltpu
from jax.experimental.pallas import tpu_sc as plsc

N = 10000
E = 320000
D = 128
HALF = 64
NSUB = 16
CHUNK = 80
ROWS_PER_SUB = E // NSUB // CHUNK
NODE_PER_SUB = 624
WB = 208
TAIL0 = NSUB * NODE_PER_SUB

_mesh = plsc.VectorSubcoreMesh(core_axis_name="c", subcore_axis_name="s")


@functools.partial(
    pl.kernel,
    mesh=_mesh,
    out_type=jax.ShapeDtypeStruct((2, N, HALF), jnp.float32),
    scratch_types=[
        pltpu.VMEM((ROWS_PER_SUB, CHUNK), jnp.int32),
        pltpu.VMEM((ROWS_PER_SUB, CHUNK), jnp.int32),
        pltpu.VMEM((CHUNK, HALF), jnp.float32),
        pltpu.VMEM((CHUNK, HALF), jnp.float32),
        pltpu.VMEM((WB, HALF), jnp.float32),
        pltpu.VMEM_SHARED((N, HALF), jnp.float32),
        pltpu.SemaphoreType.DMA,
        pltpu.SemaphoreType.DMA,
    ],
    compiler_params=pltpu.CompilerParams(use_tc_tiling_on_sc=False),
)
def _sc_agg(xflat, gsrc, dstr, out, gidx, didx, buf0, buf1, bounce, acc, sem0, sem1):
    c = lax.axis_index("c")
    s = lax.axis_index("s")

    pltpu.sync_copy(gsrc.at[c, s], gidx)
    pltpu.sync_copy(dstr.at[s], didx)

    for k in range(NODE_PER_SUB // WB):
        r0 = s * NODE_PER_SUB + k * WB
        pltpu.sync_copy(xflat.at[pl.ds(c * N + r0, WB)], bounce)
        pltpu.sync_copy(bounce, acc.at[pl.ds(r0, WB)])

    @pl.when(s == 0)
    def _init_tail():
        pltpu.sync_copy(xflat.at[pl.ds(c * N + TAIL0, N - TAIL0)], bounce.at[pl.ds(0, N - TAIL0)])
        pltpu.sync_copy(bounce.at[pl.ds(0, N - TAIL0)], acc.at[pl.ds(TAIL0, N - TAIL0)])

    plsc.subcore_barrier()

    bufs = (buf0, buf1)
    sems = (sem0, sem1)

    def start(j, b):
        pltpu.make_async_copy(xflat.at[gidx.at[j]], bufs[b], sems[b]).start()

    def wait(j, b):
        pltpu.make_async_copy(xflat.at[gidx.at[j]], bufs[b], sems[b]).wait()

    start(0, 0)
    start(1, 1)

    @pl.loop(0, ROWS_PER_SUB, step=2)
    def _edge_loop(j):
        for b in range(2):
            jj = j + b
            wait(jj, b)
            pltpu.sync_copy(bufs[b], acc.at[didx.at[jj]], add=True)

            @pl.when(jj + 2 < ROWS_PER_SUB)
            def _():
                start(jj + 2, b)

    plsc.subcore_barrier()

    for k in range(NODE_PER_SUB // WB):
        r0 = s * NODE_PER_SUB + k * WB
        pltpu.sync_copy(acc.at[pl.ds(r0, WB)], bounce)
        pltpu.sync_copy(bounce, out.at[c, pl.ds(r0, WB)])

    @pl.when(s == 0)
    def _wb_tail():
        pltpu.sync_copy(acc.at[pl.ds(TAIL0, N - TAIL0)], bounce.at[pl.ds(0, N - TAIL0)])
        pltpu.sync_copy(bounce.at[pl.ds(0, N - TAIL0)], out.at[c, pl.ds(TAIL0, N - TAIL0)])


BLK = 2000


def _mlp_body(eps_ref, x_ref, lo_ref, hi_ref, w1_ref, b1_ref, w2_ref, b2_ref,
              w3_ref, b3_ref, w4_ref, b4_ref, o_ref):
    e = eps_ref[0]
    hp = jnp.concatenate([lo_ref[...], hi_ref[...]], axis=1)
    h = hp + e * x_ref[...]
    h1 = jnp.maximum(
        jnp.dot(h, w1_ref[...], preferred_element_type=jnp.float32) + b1_ref[...], 0.0)
    h2 = jnp.dot(h1, w2_ref[...], preferred_element_type=jnp.float32) + b2_ref[...]
    h3 = jnp.maximum(
        jnp.dot(h2, w3_ref[...], preferred_element_type=jnp.float32) + b3_ref[...], 0.0)
    h4 = jnp.dot(h3, w4_ref[...], preferred_element_type=jnp.float32) + b4_ref[...]
    o_ref[...] = 1.0 / (1.0 + jnp.exp(-h4))


def _mlp(eps1, x, lo, hi, W1, b1, W2, b2, W3, b3, W4, b4):
    row_spec = pl.BlockSpec((BLK, D), lambda i: (i, 0))
    half_spec = pl.BlockSpec((BLK, HALF), lambda i: (i, 0))
    w_spec = pl.BlockSpec((D, D), lambda i: (0, 0))
    b_spec = pl.BlockSpec((1, D), lambda i: (0, 0))
    return pl.pallas_call(
        _mlp_body,
        grid=(N // BLK,),
        in_specs=[
            pl.BlockSpec(memory_space=pltpu.SMEM),
            row_spec, half_spec, half_spec,
            w_spec, b_spec, w_spec, b_spec,
            w_spec, b_spec, w_spec, b_spec,
        ],
        out_specs=row_spec,
        out_shape=jax.ShapeDtypeStruct((N, D), jnp.float32),
    )(eps1, x, lo, hi, W1, b1, W2, b2, W3, b3, W4, b4)


def kernel(x, edge_index, eps, W1, b1, W2, b2, W3, b3, W4, b4):
    ei = edge_index.astype(jnp.int32)
    src = ei[0]
    gsrc = jnp.stack([src, src + N]).reshape(2, NSUB, ROWS_PER_SUB, CHUNK)
    dstr = ei[1].reshape(NSUB, ROWS_PER_SUB, CHUNK)
    xflat = x.reshape(N, 2, HALF).transpose(1, 0, 2).reshape(2 * N, HALF)
    hpart = _sc_agg(xflat, gsrc, dstr)
    eps1 = jnp.reshape(eps, (1,)).astype(jnp.float32)
    return _mlp(eps1, x, hpart[0], hpart[1],
                W1, b1.reshape(1, D), W2, b2.reshape(1, D),
                W3, b3.reshape(1, D), W4, b4.reshape(1, D))


# speedup vs baseline: 11.6689x; 1.4488x over previous
import functools

import jax
import jax.numpy as jnp
from jax import lax
from jax.experimental import pallas as pl
from jax.experimental.pallas import tpu as pltpu
from jax.experimental.pallas import tpu_sc as plsc

N = 10000
E = 320000
D = 128
HALF = 64
NSUB = 16
CHUNK = 128
CPS = 160
EPAD = NSUB * CPS * CHUNK
NDUMMY = 8
NODE_PER_SUB = 624
WB = 208
TAIL0 = NSUB * NODE_PER_SUB
NBUF = 4

_mesh = plsc.VectorSubcoreMesh(core_axis_name="c", subcore_axis_name="s")


@functools.partial(
    pl.kernel,
    mesh=_mesh,
    out_type=jax.ShapeDtypeStruct((N, D), jnp.float32),
    scratch_types=[
        pltpu.VMEM((CPS, CHUNK), jnp.int32),
        pltpu.VMEM((CPS, CHUNK), jnp.int32),
        [pltpu.VMEM((CHUNK, HALF), jnp.float32) for _ in range(NBUF)],
        pltpu.VMEM((WB, HALF), jnp.float32),
        pltpu.VMEM_SHARED((N + NDUMMY, HALF), jnp.float32),
        [pltpu.SemaphoreType.DMA for _ in range(NBUF)],
        [pltpu.SemaphoreType.DMA for _ in range(NBUF)],
    ],
    compiler_params=pltpu.CompilerParams(use_tc_tiling_on_sc=False),
)
def _sc_agg(xflat, gsrc, dstr, zeros, out, gidx, didx, bufs, bounce, acc, gsems, ssems):
    c = lax.axis_index("c")
    s = lax.axis_index("s")

    pltpu.sync_copy(gsrc.at[c, s], gidx)
    pltpu.sync_copy(dstr.at[s], didx)

    pltpu.sync_copy(zeros, bounce)
    for k in range(NODE_PER_SUB // WB):
        r0 = s * NODE_PER_SUB + k * WB
        pltpu.sync_copy(bounce, acc.at[pl.ds(r0, WB)])

    @pl.when(s == 0)
    def _zero_tail():
        pltpu.sync_copy(bounce.at[pl.ds(0, N + NDUMMY - TAIL0)],
                        acc.at[pl.ds(TAIL0, N + NDUMMY - TAIL0)])

    plsc.subcore_barrier()

    def gather_start(j, b):
        pltpu.make_async_copy(xflat.at[gidx.at[j]], bufs[b], gsems[b]).start()

    def gather_wait(j, b):
        pltpu.make_async_copy(xflat.at[gidx.at[j]], bufs[b], gsems[b]).wait()

    def scatter_start(j, b):
        pltpu.async_copy(bufs[b], acc.at[didx.at[j]], ssems[b], add=True)

    def scatter_wait(j, b):
        pltpu.make_async_copy(bufs[b], acc.at[didx.at[j]], ssems[b]).wait()

    gather_start(0, 0)
    gather_start(1, 1)

    @pl.loop(0, CPS, step=NBUF)
    def _edge_loop(j):
        for b in range(NBUF):
            jj = j + b
            gather_wait(jj, b)
            scatter_start(jj, b)

            @pl.when(jj >= 2)
            def _():
                scatter_wait(jj - 2, (b - 2) % NBUF)

            @pl.when(jj + 2 < CPS)
            def _():
                gather_start(jj + 2, (b + 2) % NBUF)

    scatter_wait(CPS - 2, (CPS - 2) % NBUF)
    scatter_wait(CPS - 1, (CPS - 1) % NBUF)

    plsc.subcore_barrier()

    for k in range(NODE_PER_SUB // WB):
        r0 = s * NODE_PER_SUB + k * WB
        pltpu.sync_copy(acc.at[pl.ds(r0, WB)], bounce)
        pltpu.sync_copy(bounce, out.at[pl.ds(r0, WB), pl.ds(c * HALF, HALF)])

    @pl.when(s == 0)
    def _wb_tail():
        pltpu.sync_copy(acc.at[pl.ds(TAIL0, N - TAIL0)], bounce.at[pl.ds(0, N - TAIL0)])
        pltpu.sync_copy(bounce.at[pl.ds(0, N - TAIL0)],
                        out.at[pl.ds(TAIL0, N - TAIL0), pl.ds(c * HALF, HALF)])


BLK = 2000


def _mlp_body(eps_ref, x_ref, agg_ref, w1_ref, b1_ref, w2_ref, b2_ref,
              w3_ref, b3_ref, w4_ref, b4_ref, o_ref):
    e = eps_ref[0]
    h = agg_ref[...] + (1.0 + e) * x_ref[...]
    h1 = jnp.maximum(
        jnp.dot(h, w1_ref[...], preferred_element_type=jnp.float32) + b1_ref[...], 0.0)
    h2 = jnp.dot(h1, w2_ref[...], preferred_element_type=jnp.float32) + b2_ref[...]
    h3 = jnp.maximum(
        jnp.dot(h2, w3_ref[...], preferred_element_type=jnp.float32) + b3_ref[...], 0.0)
    h4 = jnp.dot(h3, w4_ref[...], preferred_element_type=jnp.float32) + b4_ref[...]
    o_ref[...] = 1.0 / (1.0 + jnp.exp(-h4))


def _mlp(eps1, x, agg, W1, b1, W2, b2, W3, b3, W4, b4):
    row_spec = pl.BlockSpec((BLK, D), lambda i: (i, 0))
    w_spec = pl.BlockSpec((D, D), lambda i: (0, 0))
    b_spec = pl.BlockSpec((1, D), lambda i: (0, 0))
    return pl.pallas_call(
        _mlp_body,
        grid=(N // BLK,),
        in_specs=[
            pl.BlockSpec(memory_space=pltpu.SMEM),
            row_spec, row_spec,
            w_spec, b_spec, w_spec, b_spec,
            w_spec, b_spec, w_spec, b_spec,
        ],
        out_specs=row_spec,
        out_shape=jax.ShapeDtypeStruct((N, D), jnp.float32),
    )(eps1, x, agg, W1, b1, W2, b2, W3, b3, W4, b4)


def kernel(x, edge_index, eps, W1, b1, W2, b2, W3, b3, W4, b4):
    ei = edge_index.astype(jnp.int32)
    src = ei[0]
    dst = ei[1]
    npad = EPAD - E
    iota = lax.iota(jnp.int32, npad)
    src_all = jnp.concatenate([src, iota % N])
    dst_all = jnp.concatenate([dst, N + (iota % NDUMMY)])
    s2 = src_all * 2
    gsrc = jnp.stack([s2, s2 + 1]).reshape(2, NSUB, CPS, CHUNK)
    dstr = dst_all.reshape(NSUB, CPS, CHUNK)
    xflat = x.reshape(2 * N, HALF)
    zeros = jnp.zeros((WB, HALF), jnp.float32)
    agg = _sc_agg(xflat, gsrc, dstr, zeros)
    eps1 = jnp.reshape(eps, (1,)).astype(jnp.float32)
    return _mlp(eps1, x, agg,
                W1, b1.reshape(1, D), W2, b2.reshape(1, D),
                W3, b3.reshape(1, D), W4, b4.reshape(1, D))


# speedup vs baseline: 12.8025x; 1.0971x over previous
import functools

import jax
import jax.numpy as jnp
from jax import lax
from jax.experimental import pallas as pl
from jax.experimental.pallas import tpu as pltpu
from jax.experimental.pallas import tpu_sc as plsc

N = 10000
E = 320000
D = 128
HALF = 64
NSUB = 16
CHUNK = 128
CPS = 160
EPAD = NSUB * CPS * CHUNK
NDUMMY = 8
NODE_PER_SUB = 624
WB = 208
TAIL0 = NSUB * NODE_PER_SUB
NBUF = 4
GDEPTH = 3
SLAG = 1

_mesh = plsc.VectorSubcoreMesh(core_axis_name="c", subcore_axis_name="s")


@functools.partial(
    pl.kernel,
    mesh=_mesh,
    out_type=jax.ShapeDtypeStruct((N, D), jnp.float32),
    scratch_types=[
        pltpu.VMEM((CPS, CHUNK), jnp.int32),
        pltpu.VMEM((CPS, CHUNK), jnp.int32),
        [pltpu.VMEM((CHUNK, HALF), jnp.float32) for _ in range(NBUF)],
        pltpu.VMEM((WB, HALF), jnp.float32),
        pltpu.VMEM_SHARED((N + NDUMMY, HALF), jnp.float32),
        [pltpu.SemaphoreType.DMA for _ in range(NBUF)],
        [pltpu.SemaphoreType.DMA for _ in range(NBUF)],
    ],
    compiler_params=pltpu.CompilerParams(use_tc_tiling_on_sc=False),
)
def _sc_agg(xflat, gsrc, dstr, zeros, out, gidx, didx, bufs, bounce, acc, gsems, ssems):
    c = lax.axis_index("c")
    s = lax.axis_index("s")

    pltpu.sync_copy(gsrc.at[c, s], gidx)
    pltpu.sync_copy(dstr.at[s], didx)

    pltpu.sync_copy(zeros, bounce)
    for k in range(NODE_PER_SUB // WB):
        r0 = s * NODE_PER_SUB + k * WB
        pltpu.sync_copy(bounce, acc.at[pl.ds(r0, WB)])

    @pl.when(s == 0)
    def _zero_tail():
        pltpu.sync_copy(bounce.at[pl.ds(0, N + NDUMMY - TAIL0)],
                        acc.at[pl.ds(TAIL0, N + NDUMMY - TAIL0)])

    plsc.subcore_barrier()

    def gather_start(j, b):
        pltpu.make_async_copy(xflat.at[gidx.at[j]], bufs[b], gsems[b]).start()

    def gather_wait(j, b):
        pltpu.make_async_copy(xflat.at[gidx.at[j]], bufs[b], gsems[b]).wait()

    def scatter_start(j, b):
        pltpu.async_copy(bufs[b], acc.at[didx.at[j]], ssems[b], add=True)

    def scatter_wait(j, b):
        pltpu.make_async_copy(bufs[b], acc.at[didx.at[j]], ssems[b]).wait()

    for b0 in range(GDEPTH):
        gather_start(b0, b0)

    @pl.loop(0, CPS, step=NBUF)
    def _edge_loop(j):
        for b in range(NBUF):
            jj = j + b
            gather_wait(jj, b)
            scatter_start(jj, b)

            @pl.when(jj >= SLAG)
            def _():
                scatter_wait(jj - SLAG, (b - SLAG) % NBUF)

            @pl.when(jj + GDEPTH < CPS)
            def _():
                gather_start(jj + GDEPTH, (b + GDEPTH) % NBUF)

    for jt in range(CPS - SLAG, CPS):
        scatter_wait(jt, jt % NBUF)

    plsc.subcore_barrier()

    for k in range(NODE_PER_SUB // WB):
        r0 = s * NODE_PER_SUB + k * WB
        pltpu.sync_copy(acc.at[pl.ds(r0, WB)], bounce)
        pltpu.sync_copy(bounce, out.at[pl.ds(r0, WB), pl.ds(c * HALF, HALF)])

    @pl.when(s == 0)
    def _wb_tail():
        pltpu.sync_copy(acc.at[pl.ds(TAIL0, N - TAIL0)], bounce.at[pl.ds(0, N - TAIL0)])
        pltpu.sync_copy(bounce.at[pl.ds(0, N - TAIL0)],
                        out.at[pl.ds(TAIL0, N - TAIL0), pl.ds(c * HALF, HALF)])


BLK = 2000


def _mlp_body(eps_ref, x_ref, agg_ref, w1_ref, b1_ref, w2_ref, b2_ref,
              w3_ref, b3_ref, w4_ref, b4_ref, o_ref):
    e = eps_ref[0]
    h = agg_ref[...] + (1.0 + e) * x_ref[...]
    h1 = jnp.maximum(
        jnp.dot(h, w1_ref[...], preferred_element_type=jnp.float32) + b1_ref[...], 0.0)
    h2 = jnp.dot(h1, w2_ref[...], preferred_element_type=jnp.float32) + b2_ref[...]
    h3 = jnp.maximum(
        jnp.dot(h2, w3_ref[...], preferred_element_type=jnp.float32) + b3_ref[...], 0.0)
    h4 = jnp.dot(h3, w4_ref[...], preferred_element_type=jnp.float32) + b4_ref[...]
    o_ref[...] = 1.0 / (1.0 + jnp.exp(-h4))


def _mlp(eps1, x, agg, W1, b1, W2, b2, W3, b3, W4, b4):
    row_spec = pl.BlockSpec((BLK, D), lambda i: (i, 0))
    w_spec = pl.BlockSpec((D, D), lambda i: (0, 0))
    b_spec = pl.BlockSpec((1, D), lambda i: (0, 0))
    return pl.pallas_call(
        _mlp_body,
        grid=(N // BLK,),
        in_specs=[
            pl.BlockSpec(memory_space=pltpu.SMEM),
            row_spec, row_spec,
            w_spec, b_spec, w_spec, b_spec,
            w_spec, b_spec, w_spec, b_spec,
        ],
        out_specs=row_spec,
        out_shape=jax.ShapeDtypeStruct((N, D), jnp.float32),
    )(eps1, x, agg, W1, b1, W2, b2, W3, b3, W4, b4)


def kernel(x, edge_index, eps, W1, b1, W2, b2, W3, b3, W4, b4):
    ei = edge_index.astype(jnp.int32)
    src = ei[0]
    dst = ei[1]
    npad = EPAD - E
    iota = lax.iota(jnp.int32, npad)
    src_all = jnp.concatenate([src, iota % N])
    dst_all = jnp.concatenate([dst, N + (iota % NDUMMY)])
    s2 = src_all * 2
    gsrc = jnp.stack([s2, s2 + 1]).reshape(2, NSUB, CPS, CHUNK)
    dstr = dst_all.reshape(NSUB, CPS, CHUNK)
    xflat = x.reshape(2 * N, HALF)
    zeros = jnp.zeros((WB, HALF), jnp.float32)
    agg = _sc_agg(xflat, gsrc, dstr, zeros)
    eps1 = jnp.reshape(eps, (1,)).astype(jnp.float32)
    return _mlp(eps1, x, agg,
                W1, b1.reshape(1, D), W2, b2.reshape(1, D),
                W3, b3.reshape(1, D), W4, b4.reshape(1, D))


# speedup vs baseline: 13.4958x; 1.0542x over previous
import functools

import jax
import jax.numpy as jnp
from jax import lax
from jax.experimental import pallas as pl
from jax.experimental.pallas import tpu as pltpu
from jax.experimental.pallas import tpu_sc as plsc

N = 10000
E = 320000
D = 128
HALF = 64
NSUB = 16
CHUNK = 128
CPS = 160
EPAD = NSUB * CPS * CHUNK
NDUMMY = 8
NODE_PER_SUB = 624
WB = 104
TAIL0 = NSUB * NODE_PER_SUB
NBUF = 5
GDEPTH = 4
SLAG = 1

_mesh = plsc.VectorSubcoreMesh(core_axis_name="c", subcore_axis_name="s")


@functools.partial(
    pl.kernel,
    mesh=_mesh,
    out_type=jax.ShapeDtypeStruct((N, D), jnp.float32),
    scratch_types=[
        pltpu.VMEM((CPS, CHUNK), jnp.int32),
        pltpu.VMEM((CPS, CHUNK), jnp.int32),
        [pltpu.VMEM((CHUNK, HALF), jnp.float32) for _ in range(NBUF)],
        pltpu.VMEM((WB, HALF), jnp.float32),
        pltpu.VMEM_SHARED((N + NDUMMY, HALF), jnp.float32),
        [pltpu.SemaphoreType.DMA for _ in range(NBUF)],
        [pltpu.SemaphoreType.DMA for _ in range(NBUF)],
    ],
    compiler_params=pltpu.CompilerParams(use_tc_tiling_on_sc=False),
)
def _sc_agg(xflat, gsrc, dstr, zeros, out, gidx, didx, bufs, bounce, acc, gsems, ssems):
    c = lax.axis_index("c")
    s = lax.axis_index("s")

    pltpu.sync_copy(gsrc.at[c, s], gidx)
    pltpu.sync_copy(dstr.at[s], didx)

    pltpu.sync_copy(zeros, bounce)
    for k in range(NODE_PER_SUB // WB):
        r0 = s * NODE_PER_SUB + k * WB
        pltpu.sync_copy(bounce, acc.at[pl.ds(r0, WB)])

    @pl.when(s == 0)
    def _zero_tail():
        pltpu.sync_copy(bounce.at[pl.ds(0, N + NDUMMY - TAIL0)],
                        acc.at[pl.ds(TAIL0, N + NDUMMY - TAIL0)])

    plsc.subcore_barrier()

    def gather_start(j, b):
        pltpu.make_async_copy(xflat.at[gidx.at[j]], bufs[b], gsems[b]).start()

    def gather_wait(j, b):
        pltpu.make_async_copy(xflat.at[gidx.at[j]], bufs[b], gsems[b]).wait()

    def scatter_start(j, b):
        pltpu.async_copy(bufs[b], acc.at[didx.at[j]], ssems[b], add=True)

    def scatter_wait(j, b):
        pltpu.make_async_copy(bufs[b], acc.at[didx.at[j]], ssems[b]).wait()

    for b0 in range(GDEPTH):
        gather_start(b0, b0)

    @pl.loop(0, CPS, step=NBUF)
    def _edge_loop(j):
        for b in range(NBUF):
            jj = j + b
            gather_wait(jj, b)
            scatter_start(jj, b)

            @pl.when(jj >= SLAG)
            def _():
                scatter_wait(jj - SLAG, (b - SLAG) % NBUF)

            @pl.when(jj + GDEPTH < CPS)
            def _():
                gather_start(jj + GDEPTH, (b + GDEPTH) % NBUF)

    for jt in range(CPS - SLAG, CPS):
        scatter_wait(jt, jt % NBUF)

    plsc.subcore_barrier()

    for k in range(NODE_PER_SUB // WB):
        r0 = s * NODE_PER_SUB + k * WB
        pltpu.sync_copy(acc.at[pl.ds(r0, WB)], bounce)
        pltpu.sync_copy(bounce, out.at[pl.ds(r0, WB), pl.ds(c * HALF, HALF)])

    @pl.when(s == 0)
    def _wb_tail():
        pltpu.sync_copy(acc.at[pl.ds(TAIL0, N - TAIL0)], bounce.at[pl.ds(0, N - TAIL0)])
        pltpu.sync_copy(bounce.at[pl.ds(0, N - TAIL0)],
                        out.at[pl.ds(TAIL0, N - TAIL0), pl.ds(c * HALF, HALF)])


BLK = 2000


def _mlp_body(eps_ref, x_ref, agg_ref, w1_ref, b1_ref, w2_ref, b2_ref,
              w3_ref, b3_ref, w4_ref, b4_ref, o_ref):
    e = eps_ref[0]
    h = agg_ref[...] + (1.0 + e) * x_ref[...]
    h1 = jnp.maximum(
        jnp.dot(h, w1_ref[...], preferred_element_type=jnp.float32) + b1_ref[...], 0.0)
    h2 = jnp.dot(h1, w2_ref[...], preferred_element_type=jnp.float32) + b2_ref[...]
    h3 = jnp.maximum(
        jnp.dot(h2, w3_ref[...], preferred_element_type=jnp.float32) + b3_ref[...], 0.0)
    h4 = jnp.dot(h3, w4_ref[...], preferred_element_type=jnp.float32) + b4_ref[...]
    o_ref[...] = 1.0 / (1.0 + jnp.exp(-h4))


def _mlp(eps1, x, agg, W1, b1, W2, b2, W3, b3, W4, b4):
    row_spec = pl.BlockSpec((BLK, D), lambda i: (i, 0))
    w_spec = pl.BlockSpec((D, D), lambda i: (0, 0))
    b_spec = pl.BlockSpec((1, D), lambda i: (0, 0))
    return pl.pallas_call(
        _mlp_body,
        grid=(N // BLK,),
        in_specs=[
            pl.BlockSpec(memory_space=pltpu.SMEM),
            row_spec, row_spec,
            w_spec, b_spec, w_spec, b_spec,
            w_spec, b_spec, w_spec, b_spec,
        ],
        out_specs=row_spec,
        out_shape=jax.ShapeDtypeStruct((N, D), jnp.float32),
    )(eps1, x, agg, W1, b1, W2, b2, W3, b3, W4, b4)


def kernel(x, edge_index, eps, W1, b1, W2, b2, W3, b3, W4, b4):
    ei = edge_index.astype(jnp.int32)
    src = ei[0]
    dst = ei[1]
    npad = EPAD - E
    iota = lax.iota(jnp.int32, npad)
    src_all = jnp.concatenate([src, iota % N])
    dst_all = jnp.concatenate([dst, N + (iota % NDUMMY)])
    s2 = src_all * 2
    gsrc = jnp.stack([s2, s2 + 1]).reshape(2, NSUB, CPS, CHUNK)
    dstr = dst_all.reshape(NSUB, CPS, CHUNK)
    xflat = x.reshape(2 * N, HALF)
    zeros = jnp.zeros((WB, HALF), jnp.float32)
    agg = _sc_agg(xflat, gsrc, dstr, zeros)
    eps1 = jnp.reshape(eps, (1,)).astype(jnp.float32)
    return _mlp(eps1, x, agg,
                W1, b1.reshape(1, D), W2, b2.reshape(1, D),
                W3, b3.reshape(1, D), W4, b4.reshape(1, D))


# speedup vs baseline: 15.3843x; 1.1399x over previous
import functools

import jax
import jax.numpy as jnp
from jax import lax
from jax.experimental import pallas as pl
from jax.experimental.pallas import tpu as pltpu
from jax.experimental.pallas import tpu_sc as plsc

N = 10000
E = 320000
D = 128
HALF = 64
NSUB = 16
CHUNK = 128
CPS = 160
EPAD = NSUB * CPS * CHUNK
NDUMMY = 8
NODE_PER_SUB = 624
WB = 104
TAIL0 = NSUB * NODE_PER_SUB
NBUF = 5
GDEPTH = 4
SLAG = 1

_mesh = plsc.VectorSubcoreMesh(core_axis_name="c", subcore_axis_name="s")


@functools.partial(
    pl.kernel,
    mesh=_mesh,
    out_type=jax.ShapeDtypeStruct((N, D), jnp.float32),
    scratch_types=[
        pltpu.VMEM((CPS, CHUNK), jnp.int32),
        pltpu.VMEM((CPS, CHUNK), jnp.int32),
        [pltpu.VMEM((CHUNK, HALF), jnp.float32) for _ in range(NBUF)],
        pltpu.VMEM((WB, HALF), jnp.float32),
        pltpu.VMEM_SHARED((N + NDUMMY, HALF), jnp.float32),
        [pltpu.SemaphoreType.DMA for _ in range(NBUF)],
        [pltpu.SemaphoreType.DMA for _ in range(NBUF)],
    ],
    compiler_params=pltpu.CompilerParams(use_tc_tiling_on_sc=False),
)
def _sc_agg(xflat, comb, out, gidx, didx, bufs, bounce, acc, gsems, ssems):
    c = lax.axis_index("c")
    s = lax.axis_index("s")

    pltpu.sync_copy(comb.at[0, s], gidx)
    pltpu.sync_copy(comb.at[1, s], didx)

    zv = jnp.zeros((16,), jnp.float32)

    @pl.loop(0, WB)
    def _zrow(r):
        for q in range(HALF // 16):
            bounce[r, pl.ds(q * 16, 16)] = zv

    @pl.when(c == 1)
    def _shift_gidx():
        @pl.loop(0, CPS)
        def _grow(r):
            for q in range(CHUNK // 16):
                v = gidx[r, pl.ds(q * 16, 16)]
                gidx[r, pl.ds(q * 16, 16)] = v + 1

    for k in range(NODE_PER_SUB // WB):
        r0 = s * NODE_PER_SUB + k * WB
        pltpu.sync_copy(bounce, acc.at[pl.ds(r0, WB)])

    @pl.when(s == 0)
    def _zero_tail():
        pltpu.sync_copy(bounce.at[pl.ds(0, N + NDUMMY - TAIL0)],
                        acc.at[pl.ds(TAIL0, N + NDUMMY - TAIL0)])

    plsc.subcore_barrier()

    def gather_start(j, b):
        pltpu.make_async_copy(xflat.at[gidx.at[j]], bufs[b], gsems[b]).start()

    def gather_wait(j, b):
        pltpu.make_async_copy(xflat.at[gidx.at[j]], bufs[b], gsems[b]).wait()

    def scatter_start(j, b):
        pltpu.async_copy(bufs[b], acc.at[didx.at[j]], ssems[b], add=True)

    def scatter_wait(j, b):
        pltpu.make_async_copy(bufs[b], acc.at[didx.at[j]], ssems[b]).wait()

    for b0 in range(GDEPTH):
        gather_start(b0, b0)

    @pl.loop(0, CPS, step=NBUF)
    def _edge_loop(j):
        for b in range(NBUF):
            jj = j + b
            gather_wait(jj, b)
            scatter_start(jj, b)

            @pl.when(jj >= SLAG)
            def _():
                scatter_wait(jj - SLAG, (b - SLAG) % NBUF)

            @pl.when(jj + GDEPTH < CPS)
            def _():
                gather_start(jj + GDEPTH, (b + GDEPTH) % NBUF)

    for jt in range(CPS - SLAG, CPS):
        scatter_wait(jt, jt % NBUF)

    plsc.subcore_barrier()

    for k in range(NODE_PER_SUB // WB):
        r0 = s * NODE_PER_SUB + k * WB
        pltpu.sync_copy(acc.at[pl.ds(r0, WB)], bounce)
        pltpu.sync_copy(bounce, out.at[pl.ds(r0, WB), pl.ds(c * HALF, HALF)])

    @pl.when(s == 0)
    def _wb_tail():
        pltpu.sync_copy(acc.at[pl.ds(TAIL0, N - TAIL0)], bounce.at[pl.ds(0, N - TAIL0)])
        pltpu.sync_copy(bounce.at[pl.ds(0, N - TAIL0)],
                        out.at[pl.ds(TAIL0, N - TAIL0), pl.ds(c * HALF, HALF)])


BLK = 2000


def _mlp_body(eps_ref, x_ref, agg_ref, w1_ref, b1_ref, w2_ref, b2_ref,
              w3_ref, b3_ref, w4_ref, b4_ref, o_ref):
    e = eps_ref[0]
    h = agg_ref[...] + (1.0 + e) * x_ref[...]
    h1 = jnp.maximum(
        jnp.dot(h, w1_ref[...], preferred_element_type=jnp.float32) + b1_ref[...], 0.0)
    h2 = jnp.dot(h1, w2_ref[...], preferred_element_type=jnp.float32) + b2_ref[...]
    h3 = jnp.maximum(
        jnp.dot(h2, w3_ref[...], preferred_element_type=jnp.float32) + b3_ref[...], 0.0)
    h4 = jnp.dot(h3, w4_ref[...], preferred_element_type=jnp.float32) + b4_ref[...]
    o_ref[...] = 1.0 / (1.0 + jnp.exp(-h4))


def _mlp(eps1, x, agg, W1, b1, W2, b2, W3, b3, W4, b4):
    row_spec = pl.BlockSpec((BLK, D), lambda i: (i, 0))
    w_spec = pl.BlockSpec((D, D), lambda i: (0, 0))
    b_spec = pl.BlockSpec((1, D), lambda i: (0, 0))
    return pl.pallas_call(
        _mlp_body,
        grid=(N // BLK,),
        in_specs=[
            pl.BlockSpec(memory_space=pltpu.SMEM),
            row_spec, row_spec,
            w_spec, b_spec, w_spec, b_spec,
            w_spec, b_spec, w_spec, b_spec,
        ],
        out_specs=row_spec,
        out_shape=jax.ShapeDtypeStruct((N, D), jnp.float32),
    )(eps1, x, agg, W1, b1, W2, b2, W3, b3, W4, b4)


def kernel(x, edge_index, eps, W1, b1, W2, b2, W3, b3, W4, b4):
    ei = edge_index.astype(jnp.int32)
    npad = EPAD - E
    iota = lax.iota(jnp.int32, npad)
    pad = jnp.stack([iota % N, N + (iota % NDUMMY)])
    comb = (jnp.concatenate([ei, pad], axis=1)
            * jnp.array([[2], [1]], jnp.int32)).reshape(2, NSUB, CPS, CHUNK)
    xflat = x.reshape(2 * N, HALF)
    agg = _sc_agg(xflat, comb)
    eps1 = jnp.reshape(eps, (1,)).astype(jnp.float32)
    return _mlp(eps1, x, agg,
                W1, b1.reshape(1, D), W2, b2.reshape(1, D),
                W3, b3.reshape(1, D), W4, b4.reshape(1, D))
